# baseline (speedup 1.0000x reference)
import functools

import jax
import jax.numpy as jnp
from jax import lax
from jax.experimental import pallas as pl
from jax.experimental.pallas import tpu as pltpu
from jax.experimental.pallas import tpu_sc as plsc

N = 10000
C = 128
NC = 2
NS = 16
NW = NC * NS


def _pack_bf16(h):
    hh = h.shape[1] // 2
    lo = lax.bitcast_convert_type(h[:, :hh].astype(jnp.bfloat16), jnp.uint16)
    hi = lax.bitcast_convert_type(h[:, hh:].astype(jnp.bfloat16), jnp.uint16)
    word = lo.astype(jnp.uint32) | (hi.astype(jnp.uint32) << 16)
    return lax.bitcast_convert_type(word, jnp.int32)


def _unpack_bf16(p):
    lo = lax.bitcast_convert_type(lax.shift_left(p, 16), jnp.float32)
    hi = lax.bitcast_convert_type(
        lax.bitwise_and(p, jnp.int32(-65536)), jnp.float32)
    return lo, hi


def _pre_body(x_ref, w1at_ref, w1bt_ref, b1_ref, xa_ref, xb_ref):
    x = x_ref[...]
    xa_ref[...] = _pack_bf16(
        jnp.dot(x, w1at_ref[...], preferred_element_type=jnp.float32))
    xb_ref[...] = _pack_bf16(
        jnp.dot(x, w1bt_ref[...], preferred_element_type=jnp.float32)
        + b1_ref[...])


def _pre(x, w1at, w1bt, b1):
    bn = 2000
    grid = (N // bn,)
    return pl.pallas_call(
        _pre_body,
        grid=grid,
        in_specs=[
            pl.BlockSpec((bn, C), lambda i: (i, 0)),
            pl.BlockSpec((C, C), lambda i: (0, 0)),
            pl.BlockSpec((C, C), lambda i: (0, 0)),
            pl.BlockSpec((1, C), lambda i: (0, 0)),
        ],
        out_specs=[
            pl.BlockSpec((bn, C // 2), lambda i: (i, 0)),
            pl.BlockSpec((bn, C // 2), lambda i: (i, 0)),
        ],
        out_shape=[
            jax.ShapeDtypeStruct((N, C // 2), jnp.int32),
            jax.ShapeDtypeStruct((N, C // 2), jnp.int32),
        ],
    )(x, w1at, w1bt, b1)


def _gather2_body(rw, xa_hbm, xb_hbm, src_hbm, dst_hbm, oa_hbm, ob_hbm,
                  idxs_v, idxd_v, bufa0, bufa1, bufb0, bufb1,
                  sema0, sema1, semb0, semb1):
    w = lax.axis_index("s") * NC + lax.axis_index("c")
    r0 = w * rw
    pltpu.sync_copy(src_hbm.at[pl.ds(r0, rw)], idxs_v)
    pltpu.sync_copy(dst_hbm.at[pl.ds(r0, rw)], idxd_v)
    pltpu.async_copy(xa_hbm.at[idxs_v.at[0]], bufa0, sema0)
    pltpu.async_copy(xb_hbm.at[idxd_v.at[0]], bufb0, semb0)

    def step(k, _):
        g = 2 * k
        pltpu.async_copy(xa_hbm.at[idxs_v.at[g + 1]], bufa1, sema1)
        pltpu.async_copy(xb_hbm.at[idxd_v.at[g + 1]], bufb1, semb1)
        pltpu.make_async_copy(xa_hbm.at[idxs_v.at[0]], bufa0, sema0).wait()
        pltpu.sync_copy(bufa0, oa_hbm.at[pl.ds((r0 + g) * C, C)])
        pltpu.make_async_copy(xb_hbm.at[idxd_v.at[0]], bufb0, semb0).wait()
        pltpu.sync_copy(bufb0, ob_hbm.at[pl.ds((r0 + g) * C, C)])

        @pl.when(k + 1 < rw // 2)
        def _():
            pltpu.async_copy(xa_hbm.at[idxs_v.at[g + 2]], bufa0, sema0)
            pltpu.async_copy(xb_hbm.at[idxd_v.at[g + 2]], bufb0, semb0)

        pltpu.make_async_copy(xa_hbm.at[idxs_v.at[0]], bufa1, sema1).wait()
        pltpu.sync_copy(bufa1, oa_hbm.at[pl.ds((r0 + g + 1) * C, C)])
        pltpu.make_async_copy(xb_hbm.at[idxd_v.at[0]], bufb1, semb1).wait()
        pltpu.sync_copy(bufb1, ob_hbm.at[pl.ds((r0 + g + 1) * C, C)])
        return 0

    lax.fori_loop(0, rw // 2, step, 0)


def _sc_gather2(xa, xb, src2d, dst2d):
    rt = src2d.shape[0]
    rw = rt // NW
    ep = rt * C
    mesh = plsc.VectorSubcoreMesh(core_axis_name="c", subcore_axis_name="s")
    hw = C // 2
    return pl.kernel(
        functools.partial(_gather2_body, rw),
        out_type=[
            jax.ShapeDtypeStruct((ep, hw), jnp.int32),
            jax.ShapeDtypeStruct((ep, hw), jnp.int32),
        ],
        mesh=mesh,
        scratch_types=[
            pltpu.VMEM((rw, C), jnp.int32),
            pltpu.VMEM((rw, C), jnp.int32),
            pltpu.VMEM((C, hw), jnp.int32),
            pltpu.VMEM((C, hw), jnp.int32),
            pltpu.VMEM((C, hw), jnp.int32),
            pltpu.VMEM((C, hw), jnp.int32),
            pltpu.SemaphoreType.DMA,
            pltpu.SemaphoreType.DMA,
            pltpu.SemaphoreType.DMA,
            pltpu.SemaphoreType.DMA,
        ],
        compiler_params=pltpu.CompilerParams(use_tc_tiling_on_sc=False),
    )(xa, xb, src2d, dst2d)


def _mlp_body(be, gaq_ref, gbq_ref, attr_ref, w2l_ref, w2h_ref, j2_ref,
              j2b_ref, b2_ref, g1l_ref, g1h_ref, be1l_ref, be1h_ref,
              g2_ref, be2_ref, out_ref):
    f32 = jnp.float32
    j2 = j2_ref[...]
    j2b = j2b_ref[...]

    gal, gah = _unpack_bf16(gaq_ref[...])
    gbl, gbh = _unpack_bf16(gbq_ref[...])
    hl = gal + gbl
    hh = gah + gbh
    m = jnp.dot(hl + hh, j2, preferred_element_type=f32)
    dl = hl - m
    dh = hh - m
    v = jnp.dot(dl * dl + dh * dh, j2, preferred_element_type=f32)
    inv = lax.rsqrt(v + 1e-5)
    nl = dl * inv * g1l_ref[...] + be1l_ref[...]
    nh = dh * inv * g1h_ref[...] + be1h_ref[...]
    nl = jnp.maximum(nl, 0.2 * nl)
    nh = jnp.maximum(nh, 0.2 * nh)
    h2 = (jnp.dot(nl, w2l_ref[...], preferred_element_type=f32)
          + jnp.dot(nh, w2h_ref[...], preferred_element_type=f32)
          + b2_ref[...])
    m2 = jnp.dot(h2, j2b, preferred_element_type=f32)
    d = h2 - m2
    v2 = jnp.dot(d * d, j2b, preferred_element_type=f32)
    h2 = d * lax.rsqrt(v2 + 1e-5) * g2_ref[...] + be2_ref[...]
    h2 = jnp.maximum(h2, 0.2 * h2)
    at = attr_ref[0]
    hw = C // 2
    for s in range(be // C):
        hs = h2[s * hw:(s + 1) * hw, :]
        d2 = jnp.concatenate([hs[:, :C], hs[:, C:]], axis=0)
        out_ref[pl.ds(s * C, C), :] = d2 * at[:, s:s + 1]


def _mlp(gaq, gbq, attr_t, consts):
    ep2 = gaq.shape[0]
    be = 2048
    rb = be // C
    grid = (2 * ep2 // be,)
    vec = lambda i: (0, 0)
    return pl.pallas_call(
        functools.partial(_mlp_body, be),
        grid=grid,
        in_specs=[
            pl.BlockSpec((be // 2, C), lambda i: (i, 0)),
            pl.BlockSpec((be // 2, C), lambda i: (i, 0)),
            pl.BlockSpec((1, C, rb), lambda i: (i, 0, 0)),
            pl.BlockSpec((C, 2 * C), vec),
            pl.BlockSpec((C, 2 * C), vec),
            pl.BlockSpec((C, C), vec),
            pl.BlockSpec((2 * C, 2 * C), vec),
            pl.BlockSpec((1, 2 * C), vec),
            pl.BlockSpec((1, C), vec),
            pl.BlockSpec((1, C), vec),
            pl.BlockSpec((1, C), vec),
            pl.BlockSpec((1, C), vec),
            pl.BlockSpec((1, 2 * C), vec),
            pl.BlockSpec((1, 2 * C), vec),
        ],
        out_specs=pl.BlockSpec((be, C), lambda i: (i, 0)),
        out_shape=jax.ShapeDtypeStruct((2 * ep2, C), jnp.float32),
    )(gaq, gbq, attr_t, *consts)


def _scatter_body(rw, nchunks, *refs):
    (ef_and_dst, (out_hbm,), (idx_v, bufa, bufb, zbuf, accum, sema, semb)) = (
        refs[: 2 * nchunks], refs[2 * nchunks: 2 * nchunks + 1],
        refs[2 * nchunks + 1:])
    ef_refs = ef_and_dst[:nchunks]
    dst_refs = ef_and_dst[nchunks:]
    c = lax.axis_index("c")
    s = lax.axis_index("s")
    w = s * NC + c
    stripe = 624
    off0 = s * stripe
    zr = 16

    zero16 = jnp.zeros((16,), jnp.float32)

    def zstep(k, _):
        i = k // (C // 16)
        j = k % (C // 16)
        zbuf[i, pl.ds(j * 16, 16)] = zero16
        return 0

    lax.fori_loop(0, zr * (C // 16), zstep, 0)

    nblk = jnp.where(s < NS - 1, stripe // zr, (N - (NS - 1) * stripe) // zr)

    def zcopy(j, _):
        pltpu.sync_copy(zbuf, accum.at[pl.ds(off0 + j * zr, zr)])
        return 0

    lax.fori_loop(0, nblk, zcopy, 0)
    plsc.subcore_barrier()

    r0 = w * rw
    for ef_hbm, dst_hbm in zip(ef_refs, dst_refs):
        pltpu.sync_copy(dst_hbm.at[pl.ds(r0, rw)], idx_v)
        pltpu.async_copy(ef_hbm.at[pl.ds(r0 * C, C)], bufa, sema)

        def step(k, _, ef_hbm=ef_hbm):
            g = 2 * k
            pltpu.async_copy(ef_hbm.at[pl.ds((r0 + g + 1) * C, C)], bufb, semb)
            pltpu.make_async_copy(ef_hbm.at[pl.ds(0, C)], bufa, sema).wait()
            pltpu.sync_copy(bufa, accum.at[idx_v.at[g]], add=True)

            @pl.when(k + 1 < rw // 2)
            def _():
                pltpu.async_copy(ef_hbm.at[pl.ds((r0 + g + 2) * C, C)], bufa, sema)

            pltpu.make_async_copy(ef_hbm.at[pl.ds(0, C)], bufb, semb).wait()
            pltpu.sync_copy(bufb, accum.at[idx_v.at[g + 1]], add=True)
            return 0

        lax.fori_loop(0, rw // 2, step, 0)
    plsc.subcore_barrier()

    @pl.when(s < NS - 1)
    def _():
        pltpu.sync_copy(
            accum.at[pl.ds(off0, stripe)],
            out_hbm.at[c, pl.ds(off0, stripe)],
        )

    @pl.when(s == NS - 1)
    def _():
        last0 = (NS - 1) * stripe
        pltpu.sync_copy(
            accum.at[pl.ds(last0, N - last0)],
            out_hbm.at[c, pl.ds(last0, N - last0)],
        )


def _sc_scatter(efs, dsts):
    rt = dsts[0].shape[0]
    rw = rt // NW
    nchunks = len(efs)
    mesh = plsc.VectorSubcoreMesh(core_axis_name="c", subcore_axis_name="s")
    return pl.kernel(
        functools.partial(_scatter_body, rw, nchunks),
        out_type=jax.ShapeDtypeStruct((NC, N, C), jnp.float32),
        mesh=mesh,
        scratch_types=[
            pltpu.VMEM((rw, C), jnp.int32),
            pltpu.VMEM((C, C), jnp.float32),
            pltpu.VMEM((C, C), jnp.float32),
            pltpu.VMEM((16, C), jnp.float32),
            pltpu.VMEM_SHARED((N, C), jnp.float32),
            pltpu.SemaphoreType.DMA,
            pltpu.SemaphoreType.DMA,
        ],
    )(*efs, *dsts)


def _sum_body(*refs):
    out_ref = refs[-1]
    acc = 0
    for p in refs[:-1]:
        acc = acc + p[0] + p[1]
    out_ref[...] = acc


def _psum(parts):
    bn = 2000
    return pl.pallas_call(
        _sum_body,
        grid=(N // bn,),
        in_specs=[pl.BlockSpec((NC, bn, C), lambda i: (0, i, 0))
                  for _ in parts],
        out_specs=pl.BlockSpec((bn, C), lambda i: (i, 0)),
        out_shape=jax.ShapeDtypeStruct((N, C), jnp.float32),
    )(*parts)


def kernel(x, edge_index, edge_attr, W1, b1, g1, be1, W2, b2, g2, be2):
    e = edge_attr.shape[0]
    rt = -(-e // C)
    rt = -(-rt // (2 * NW)) * (2 * NW)
    ep = rt * C
    pad = ep - e

    n = x.shape[0]
    padidx = (jnp.arange(pad, dtype=jnp.int32)) % n
    src2d = jnp.concatenate([edge_index[0], padidx]).reshape(rt, C)
    dst2d = jnp.concatenate([edge_index[1], padidx]).reshape(rt, C)
    attrp = jnp.concatenate([edge_attr, jnp.zeros((pad,), jnp.float32)])
    attr_t = jnp.transpose(
        jnp.transpose(
            attrp.reshape(ep // 2048, 2048 // C, C // 2, 2), (0, 1, 3, 2)
        ).reshape(ep // 2048, 2048 // C, C),
        (0, 2, 1))
    dst_s2d = jnp.transpose(
        jnp.concatenate([edge_index[1], padidx]).reshape(rt, C // 2, 2),
        (0, 2, 1)).reshape(rt, C)

    hw = C // 2
    w1at = W1[:, :C].T
    w1bt = W1[:, C:].T
    w2t = W2.T
    b1r = b1.reshape(1, C)
    zc = jnp.zeros((hw, C), jnp.float32)
    w2l = jnp.concatenate([
        jnp.concatenate([w2t[:hw], zc], axis=1),
        jnp.concatenate([zc, w2t[:hw]], axis=1)], axis=0)
    w2h = jnp.concatenate([
        jnp.concatenate([w2t[hw:], zc], axis=1),
        jnp.concatenate([zc, w2t[hw:]], axis=1)], axis=0)
    blk = jnp.arange(C) // hw
    j2 = jnp.where(blk[:, None] == blk[None, :], 1.0 / C, 0.0)
    blk2 = jnp.arange(2 * C) // C
    j2b = jnp.where(blk2[:, None] == blk2[None, :], 1.0 / C, 0.0)
    tile2 = lambda a: jnp.concatenate([a, a]).reshape(1, -1)
    consts = (
        w2l, w2h, j2, j2b,
        tile2(b2),
        tile2(g1[:hw]), tile2(g1[hw:]),
        tile2(be1[:hw]), tile2(be1[hw:]),
        tile2(g2), tile2(be2),
    )

    xa, xb = _pre(x, w1at, w1bt, b1r)

    k_chunks = 5
    rc = rt // k_chunks
    blocks_per_chunk = attr_t.shape[0] // k_chunks
    efs, dsts = [], []
    for k in range(k_chunks):
        srck = src2d[k * rc:(k + 1) * rc]
        dstk = dst2d[k * rc:(k + 1) * rc]
        attrk = attr_t[k * blocks_per_chunk:(k + 1) * blocks_per_chunk]
        ga, gb = _sc_gather2(xa, xb, srck, dstk)
        gaq = ga.reshape(ga.shape[0] // 2, C)
        gbq = gb.reshape(gb.shape[0] // 2, C)
        efs.append(_mlp(gaq, gbq, attrk, consts))
        dsts.append(dst_s2d[k * rc:(k + 1) * rc])

    parts = [
        _sc_scatter(efs[:2], dsts[:2]),
        _sc_scatter(efs[2:4], dsts[2:4]),
        _sc_scatter(efs[4:], dsts[4:]),
    ]
    return _psum(parts)


# speedup vs baseline: 1.6378x; 1.6378x over previous
import functools

import jax
import jax.numpy as jnp
from jax import lax
from jax.experimental import pallas as pl
from jax.experimental.pallas import tpu as pltpu
from jax.experimental.pallas import tpu_sc as plsc

N = 10000
C = 128
NC = 2
NS = 16
NW = NC * NS


def _pack_bf16(h):
    hh = h.shape[1] // 2
    lo = lax.bitcast_convert_type(h[:, :hh].astype(jnp.bfloat16), jnp.uint16)
    hi = lax.bitcast_convert_type(h[:, hh:].astype(jnp.bfloat16), jnp.uint16)
    word = lo.astype(jnp.uint32) | (hi.astype(jnp.uint32) << 16)
    return lax.bitcast_convert_type(word, jnp.int32)


def _unpack_bf16(p):
    lo = lax.bitcast_convert_type(lax.shift_left(p, 16), jnp.float32)
    hi = lax.bitcast_convert_type(
        lax.bitwise_and(p, jnp.int32(-65536)), jnp.float32)
    return lo, hi


def _pre_body(x_ref, w1at_ref, w1bt_ref, b1_ref, xa_ref, xb_ref):
    x = x_ref[...]
    xa_ref[...] = _pack_bf16(
        jnp.dot(x, w1at_ref[...], preferred_element_type=jnp.float32))
    xb_ref[...] = _pack_bf16(
        jnp.dot(x, w1bt_ref[...], preferred_element_type=jnp.float32)
        + b1_ref[...])


def _pre(x, w1at, w1bt, b1):
    bn = 2000
    grid = (N // bn,)
    return pl.pallas_call(
        _pre_body,
        grid=grid,
        in_specs=[
            pl.BlockSpec((bn, C), lambda i: (i, 0)),
            pl.BlockSpec((C, C), lambda i: (0, 0)),
            pl.BlockSpec((C, C), lambda i: (0, 0)),
            pl.BlockSpec((1, C), lambda i: (0, 0)),
        ],
        out_specs=[
            pl.BlockSpec((bn, C // 2), lambda i: (i, 0)),
            pl.BlockSpec((bn, C // 2), lambda i: (i, 0)),
        ],
        out_shape=[
            jax.ShapeDtypeStruct((N, C // 2), jnp.int32),
            jax.ShapeDtypeStruct((N, C // 2), jnp.int32),
        ],
    )(x, w1at, w1bt, b1)


def _gather2_body(rw, xa_hbm, xb_hbm, src_hbm, dst_hbm, oa_hbm, ob_hbm,
                  idxs_v, idxd_v, bufa0, bufa1, bufb0, bufb1,
                  sema0, sema1, semb0, semb1):
    w = lax.axis_index("s") * NC + lax.axis_index("c")
    r0 = w * rw
    pltpu.sync_copy(src_hbm.at[pl.ds(r0, rw)], idxs_v)
    pltpu.sync_copy(dst_hbm.at[pl.ds(r0, rw)], idxd_v)
    pltpu.async_copy(xa_hbm.at[idxs_v.at[0]], bufa0, sema0)
    pltpu.async_copy(xb_hbm.at[idxd_v.at[0]], bufb0, semb0)

    def step(k, _):
        g = 2 * k
        pltpu.async_copy(xa_hbm.at[idxs_v.at[g + 1]], bufa1, sema1)
        pltpu.async_copy(xb_hbm.at[idxd_v.at[g + 1]], bufb1, semb1)
        pltpu.make_async_copy(xa_hbm.at[idxs_v.at[0]], bufa0, sema0).wait()
        pltpu.sync_copy(bufa0, oa_hbm.at[pl.ds((r0 + g) * C, C)])
        pltpu.make_async_copy(xb_hbm.at[idxd_v.at[0]], bufb0, semb0).wait()
        pltpu.sync_copy(bufb0, ob_hbm.at[pl.ds((r0 + g) * C, C)])

        @pl.when(k + 1 < rw // 2)
        def _():
            pltpu.async_copy(xa_hbm.at[idxs_v.at[g + 2]], bufa0, sema0)
            pltpu.async_copy(xb_hbm.at[idxd_v.at[g + 2]], bufb0, semb0)

        pltpu.make_async_copy(xa_hbm.at[idxs_v.at[0]], bufa1, sema1).wait()
        pltpu.sync_copy(bufa1, oa_hbm.at[pl.ds((r0 + g + 1) * C, C)])
        pltpu.make_async_copy(xb_hbm.at[idxd_v.at[0]], bufb1, semb1).wait()
        pltpu.sync_copy(bufb1, ob_hbm.at[pl.ds((r0 + g + 1) * C, C)])
        return 0

    lax.fori_loop(0, rw // 2, step, 0)


def _sc_gather2(xa, xb, src2d, dst2d):
    rt = src2d.shape[0]
    rw = rt // NW
    ep = rt * C
    mesh = plsc.VectorSubcoreMesh(core_axis_name="c", subcore_axis_name="s")
    hw = C // 2
    return pl.kernel(
        functools.partial(_gather2_body, rw),
        out_type=[
            jax.ShapeDtypeStruct((ep, hw), jnp.int32),
            jax.ShapeDtypeStruct((ep, hw), jnp.int32),
        ],
        mesh=mesh,
        scratch_types=[
            pltpu.VMEM((rw, C), jnp.int32),
            pltpu.VMEM((rw, C), jnp.int32),
            pltpu.VMEM((C, hw), jnp.int32),
            pltpu.VMEM((C, hw), jnp.int32),
            pltpu.VMEM((C, hw), jnp.int32),
            pltpu.VMEM((C, hw), jnp.int32),
            pltpu.SemaphoreType.DMA,
            pltpu.SemaphoreType.DMA,
            pltpu.SemaphoreType.DMA,
            pltpu.SemaphoreType.DMA,
        ],
        compiler_params=pltpu.CompilerParams(use_tc_tiling_on_sc=False),
    )(xa, xb, src2d, dst2d)


def _mlp_body(be, gaq_ref, gbq_ref, attr_ref, w2l_ref, w2h_ref, j2_ref,
              j2b_ref, b2_ref, g1l_ref, g1h_ref, be1l_ref, be1h_ref,
              g2_ref, be2_ref, out_ref):
    f32 = jnp.float32
    j2 = j2_ref[...]
    j2b = j2b_ref[...]

    gal, gah = _unpack_bf16(gaq_ref[...])
    gbl, gbh = _unpack_bf16(gbq_ref[...])
    hl = gal + gbl
    hh = gah + gbh
    m = jnp.dot(hl + hh, j2, preferred_element_type=f32)
    dl = hl - m
    dh = hh - m
    v = jnp.dot(dl * dl + dh * dh, j2, preferred_element_type=f32)
    inv = lax.rsqrt(v + 1e-5)
    nl = dl * inv * g1l_ref[...] + be1l_ref[...]
    nh = dh * inv * g1h_ref[...] + be1h_ref[...]
    nl = jnp.maximum(nl, 0.2 * nl)
    nh = jnp.maximum(nh, 0.2 * nh)
    h2 = (jnp.dot(nl, w2l_ref[...], preferred_element_type=f32)
          + jnp.dot(nh, w2h_ref[...], preferred_element_type=f32)
          + b2_ref[...])
    m2 = jnp.dot(h2, j2b, preferred_element_type=f32)
    d = h2 - m2
    v2 = jnp.dot(d * d, j2b, preferred_element_type=f32)
    h2 = d * lax.rsqrt(v2 + 1e-5) * g2_ref[...] + be2_ref[...]
    h2 = jnp.maximum(h2, 0.2 * h2)
    at = attr_ref[0]
    hw = C // 2
    for s in range(be // C):
        hs = h2[s * hw:(s + 1) * hw, :]
        d2 = jnp.concatenate([hs[:, :C], hs[:, C:]], axis=0)
        out_ref[pl.ds(s * C, C), :] = d2 * at[:, s:s + 1]


def _mlp(gaq, gbq, attr_t, consts):
    ep2 = gaq.shape[0]
    be = 2048
    rb = be // C
    grid = (2 * ep2 // be,)
    vec = lambda i: (0, 0)
    return pl.pallas_call(
        functools.partial(_mlp_body, be),
        grid=grid,
        in_specs=[
            pl.BlockSpec((be // 2, C), lambda i: (i, 0)),
            pl.BlockSpec((be // 2, C), lambda i: (i, 0)),
            pl.BlockSpec((1, C, rb), lambda i: (i, 0, 0)),
            pl.BlockSpec((C, 2 * C), vec),
            pl.BlockSpec((C, 2 * C), vec),
            pl.BlockSpec((C, C), vec),
            pl.BlockSpec((2 * C, 2 * C), vec),
            pl.BlockSpec((1, 2 * C), vec),
            pl.BlockSpec((1, C), vec),
            pl.BlockSpec((1, C), vec),
            pl.BlockSpec((1, C), vec),
            pl.BlockSpec((1, C), vec),
            pl.BlockSpec((1, 2 * C), vec),
            pl.BlockSpec((1, 2 * C), vec),
        ],
        out_specs=pl.BlockSpec((be, C), lambda i: (i, 0)),
        out_shape=jax.ShapeDtypeStruct((2 * ep2, C), jnp.float32),
    )(gaq, gbq, attr_t, *consts)


def _scatter_body(rw, nchunks, *refs):
    (ef_and_dst, (out_hbm,), (idx_v, bufa, bufb, zbuf, accum, sema, semb)) = (
        refs[: 2 * nchunks], refs[2 * nchunks: 2 * nchunks + 1],
        refs[2 * nchunks + 1:])
    ef_refs = ef_and_dst[:nchunks]
    dst_refs = ef_and_dst[nchunks:]
    c = lax.axis_index("c")
    s = lax.axis_index("s")
    w = s * NC + c
    stripe = 624
    off0 = s * stripe
    zr = 16

    zero16 = jnp.zeros((16,), jnp.float32)

    def zstep(k, _):
        i = k // (C // 16)
        j = k % (C // 16)
        zbuf[i, pl.ds(j * 16, 16)] = zero16
        return 0

    lax.fori_loop(0, zr * (C // 16), zstep, 0)

    nblk = jnp.where(s < NS - 1, stripe // zr, (N - (NS - 1) * stripe) // zr)

    def zcopy(j, _):
        pltpu.sync_copy(zbuf, accum.at[pl.ds(off0 + j * zr, zr)])
        return 0

    lax.fori_loop(0, nblk, zcopy, 0)
    plsc.subcore_barrier()

    r0 = w * rw
    for ef_hbm, dst_hbm in zip(ef_refs, dst_refs):
        pltpu.sync_copy(dst_hbm.at[pl.ds(r0, rw)], idx_v)
        pltpu.async_copy(ef_hbm.at[pl.ds(r0 * C, C)], bufa, sema)

        def step(k, _, ef_hbm=ef_hbm):
            g = 2 * k
            pltpu.async_copy(ef_hbm.at[pl.ds((r0 + g + 1) * C, C)], bufb, semb)
            pltpu.make_async_copy(ef_hbm.at[pl.ds(0, C)], bufa, sema).wait()
            pltpu.sync_copy(bufa, accum.at[idx_v.at[g]], add=True)

            @pl.when(k + 1 < rw // 2)
            def _():
                pltpu.async_copy(ef_hbm.at[pl.ds((r0 + g + 2) * C, C)], bufa, sema)

            pltpu.make_async_copy(ef_hbm.at[pl.ds(0, C)], bufb, semb).wait()
            pltpu.sync_copy(bufb, accum.at[idx_v.at[g + 1]], add=True)
            return 0

        lax.fori_loop(0, rw // 2, step, 0)
    plsc.subcore_barrier()

    @pl.when(s < NS - 1)
    def _():
        pltpu.sync_copy(
            accum.at[pl.ds(off0, stripe)],
            out_hbm.at[c, pl.ds(off0, stripe)],
        )

    @pl.when(s == NS - 1)
    def _():
        last0 = (NS - 1) * stripe
        pltpu.sync_copy(
            accum.at[pl.ds(last0, N - last0)],
            out_hbm.at[c, pl.ds(last0, N - last0)],
        )


def _sc_scatter(efs, dsts):
    rt = dsts[0].shape[0]
    rw = rt // NW
    nchunks = len(efs)
    mesh = plsc.VectorSubcoreMesh(core_axis_name="c", subcore_axis_name="s")
    return pl.kernel(
        functools.partial(_scatter_body, rw, nchunks),
        out_type=jax.ShapeDtypeStruct((NC, N, C), jnp.float32),
        mesh=mesh,
        scratch_types=[
            pltpu.VMEM((rw, C), jnp.int32),
            pltpu.VMEM((C, C), jnp.float32),
            pltpu.VMEM((C, C), jnp.float32),
            pltpu.VMEM((16, C), jnp.float32),
            pltpu.VMEM_SHARED((N, C), jnp.float32),
            pltpu.SemaphoreType.DMA,
            pltpu.SemaphoreType.DMA,
        ],
    )(*efs, *dsts)


def _sum_body(*refs):
    out_ref = refs[-1]
    acc = 0
    for p in refs[:-1]:
        acc = acc + p[0] + p[1]
    out_ref[...] = acc


def _psum(parts):
    bn = 2000
    return pl.pallas_call(
        _sum_body,
        grid=(N // bn,),
        in_specs=[pl.BlockSpec((NC, bn, C), lambda i: (0, i, 0))
                  for _ in parts],
        out_specs=pl.BlockSpec((bn, C), lambda i: (i, 0)),
        out_shape=jax.ShapeDtypeStruct((N, C), jnp.float32),
    )(*parts)


def kernel(x, edge_index, edge_attr, W1, b1, g1, be1, W2, b2, g2, be2):
    e = edge_attr.shape[0]
    rt = -(-e // C)
    rt = -(-rt // (2 * NW)) * (2 * NW)
    ep = rt * C
    pad = ep - e

    n = x.shape[0]
    padidx = (jnp.arange(pad, dtype=jnp.int32)) % n
    src2d = jnp.concatenate([edge_index[0], padidx]).reshape(rt, C)
    dst2d = jnp.concatenate([edge_index[1], padidx]).reshape(rt, C)
    perm = jnp.concatenate([jnp.arange(0, C, 2), jnp.arange(1, C, 2)])
    attrp = jnp.concatenate([edge_attr, jnp.zeros((pad,), jnp.float32)])
    attr_t = jnp.transpose(
        attrp.reshape(rt, C)[:, perm].reshape(ep // 2048, 2048 // C, C),
        (0, 2, 1))
    dst_s2d = dst2d[:, perm]

    hw = C // 2
    w1at = W1[:, :C].T
    w1bt = W1[:, C:].T
    w2t = W2.T
    b1r = b1.reshape(1, C)
    zc = jnp.zeros((hw, C), jnp.float32)
    w2l = jnp.concatenate([
        jnp.concatenate([w2t[:hw], zc], axis=1),
        jnp.concatenate([zc, w2t[:hw]], axis=1)], axis=0)
    w2h = jnp.concatenate([
        jnp.concatenate([w2t[hw:], zc], axis=1),
        jnp.concatenate([zc, w2t[hw:]], axis=1)], axis=0)
    blk = jnp.arange(C) // hw
    j2 = jnp.where(blk[:, None] == blk[None, :], 1.0 / C, 0.0)
    blk2 = jnp.arange(2 * C) // C
    j2b = jnp.where(blk2[:, None] == blk2[None, :], 1.0 / C, 0.0)
    tile2 = lambda a: jnp.concatenate([a, a]).reshape(1, -1)
    consts = (
        w2l, w2h, j2, j2b,
        tile2(b2),
        tile2(g1[:hw]), tile2(g1[hw:]),
        tile2(be1[:hw]), tile2(be1[hw:]),
        tile2(g2), tile2(be2),
    )

    xa, xb = _pre(x, w1at, w1bt, b1r)

    k_chunks = 5
    rc = rt // k_chunks
    blocks_per_chunk = attr_t.shape[0] // k_chunks
    efs, dsts = [], []
    for k in range(k_chunks):
        srck = src2d[k * rc:(k + 1) * rc]
        dstk = dst2d[k * rc:(k + 1) * rc]
        attrk = attr_t[k * blocks_per_chunk:(k + 1) * blocks_per_chunk]
        ga, gb = _sc_gather2(xa, xb, srck, dstk)
        gaq = ga.reshape(ga.shape[0] // 2, C)
        gbq = gb.reshape(gb.shape[0] // 2, C)
        efs.append(_mlp(gaq, gbq, attrk, consts))
        dsts.append(dst_s2d[k * rc:(k + 1) * rc])

    parts = [
        _sc_scatter(efs[:2], dsts[:2]),
        _sc_scatter(efs[2:4], dsts[2:4]),
        _sc_scatter(efs[4:], dsts[4:]),
    ]
    return _psum(parts)
